# baseline (speedup 1.0000x reference)
import functools

import jax
import jax.numpy as jnp
from jax import lax
from jax.experimental import pallas as pl
from jax.experimental.pallas import tpu as pltpu
from jax.experimental.pallas import tpu_sc as plsc

NC, NS, L = 2, 16, 16
D = 128
CHUNK = 128
N_NODES = 10000
N_ACC = 10240
ROWS_PER_TILE = N_ACC // NS
E_RAW = 320000
E_PAD = 32 * 80 * CHUNK
N_CROWS = E_PAD // CHUNK
TILE_CHUNKS = N_CROWS // NS
NBUF = 2
SBUF = 4

_MESH = plsc.VectorSubcoreMesh(core_axis_name="c", subcore_axis_name="s")
_SC_PARAMS = pltpu.CompilerParams(needs_layout_passes=False)


def _make_sc_layer(with_counts):
    if with_counts:
        out_type = [jax.ShapeDtypeStruct((NC, N_ACC, D), jnp.float32),
                    jax.ShapeDtypeStruct((NC, NS, N_ACC), jnp.float32)]
    else:
        out_type = jax.ShapeDtypeStruct((NC, N_ACC, D), jnp.float32)

    scratch = [
        pltpu.VMEM_SHARED((N_ACC, D), jnp.float32),
        pltpu.VMEM((SBUF, CHUNK), jnp.int32),
        pltpu.VMEM((NBUF, CHUNK), jnp.int32),
        [pltpu.VMEM((CHUNK, D), jnp.float32) for _ in range(NBUF)],
        [pltpu.SemaphoreType.DMA for _ in range(NBUF)],
        [pltpu.SemaphoreType.DMA for _ in range(NBUF)],
        [pltpu.SemaphoreType.DMA for _ in range(SBUF)],
    ]
    if with_counts:
        scratch.append(pltpu.VMEM((N_ACC,), jnp.float32))

    @functools.partial(
        pl.kernel, mesh=_MESH, out_type=out_type, scratch_types=scratch,
        compiler_params=_SC_PARAMS,
        name="sc_layer_cnt" if with_counts else "sc_layer",
    )
    def k(*args):
        if with_counts:
            (x_hbm, src_hbm, dst_hbm, zeros_hbm, acc_out, cnt_out,
             acc_sp, src_ring, dst_ring, rows, gsems, dsems, ssems,
             cnt_v) = args
        else:
            (x_hbm, src_hbm, dst_hbm, zeros_hbm, acc_out,
             acc_sp, src_ring, dst_ring, rows, gsems, dsems, ssems) = args
        cid = lax.axis_index("c")
        sid = lax.axis_index("s")
        row0 = sid * ROWS_PER_TILE
        crow0 = sid * TILE_CHUNKS

        pltpu.sync_copy(zeros_hbm.at[pl.ds(row0, ROWS_PER_TILE)],
                        acc_sp.at[pl.ds(row0, ROWS_PER_TILE)])
        if with_counts:
            def zbody(i, c):
                cnt_v[pl.ds(i * L, L)] = jnp.zeros((L,), jnp.float32)
                return c
            lax.fori_loop(0, N_ACC // L, zbody, 0)
        plsc.subcore_barrier()

        for s in range(NBUF):
            pltpu.sync_copy(src_hbm.at[cid, crow0 + s], src_ring.at[s])
        for s in range(NBUF, SBUF):
            pltpu.async_copy(src_hbm.at[cid, crow0 + s], src_ring.at[s],
                             ssems[s])
        for b in range(NBUF):
            pltpu.async_copy(dst_hbm.at[cid, crow0 + b], dst_ring.at[b],
                             dsems[b])
            pltpu.async_copy(x_hbm.at[src_ring.at[b]], rows[b], gsems[b])

        def outer(o, c):
            base = o * SBUF
            for s in range(SBUF):
                i = base + s
                b = s % NBUF
                pltpu.make_async_copy(dst_hbm.at[cid, crow0 + i],
                                      dst_ring.at[b], dsems[b]).wait()
                pltpu.make_async_copy(x_hbm.at[src_ring.at[s]], rows[b],
                                      gsems[b]).wait()
                pltpu.sync_copy(rows[b], acc_sp.at[dst_ring.at[b]], add=True)
                if with_counts:
                    def cbody(j, cc):
                        dv = dst_ring[b, pl.ds(j * L, L)]
                        plsc.addupdate_scatter(cnt_v, [dv],
                                               jnp.ones((L,), jnp.float32))
                        return cc
                    lax.fori_loop(0, CHUNK // L, cbody, 0)

                @pl.when(i + NBUF < TILE_CHUNKS)
                def _():
                    nxt = i + NBUF
                    sn = (s + NBUF) % SBUF
                    pltpu.make_async_copy(src_hbm.at[cid, crow0 + nxt],
                                          src_ring.at[sn], ssems[sn]).wait()
                    pltpu.async_copy(x_hbm.at[src_ring.at[sn]],
                                     rows[b], gsems[b])
                    pltpu.async_copy(dst_hbm.at[cid, crow0 + nxt],
                                     dst_ring.at[b], dsems[b])

                @pl.when(i + SBUF < TILE_CHUNKS)
                def _():
                    pltpu.async_copy(src_hbm.at[cid, crow0 + i + SBUF],
                                     src_ring.at[s], ssems[s])
            return c

        lax.fori_loop(0, TILE_CHUNKS // SBUF, outer, 0)
        plsc.subcore_barrier()

        pltpu.sync_copy(acc_sp.at[pl.ds(row0, ROWS_PER_TILE)],
                        acc_out.at[cid, pl.ds(row0, ROWS_PER_TILE)])
        if with_counts:
            pltpu.sync_copy(cnt_v, cnt_out.at[cid, sid])

    return k


_sc_layer_cnt = _make_sc_layer(True)
_sc_layer = _make_sc_layer(False)

RB = 1280


def _dense_body(acc_ref, cnt_ref, xd_ref, wn_ref, bn_ref, wr_ref,
                g_ref, b_ref, out_ref):
    i = pl.program_id(1)
    s = acc_ref[0]
    c = jnp.sum(cnt_ref[0, :, pl.ds(i * RB, RB)], axis=0)
    inv = (1.0 / jnp.maximum(c, 1.0)).reshape(RB, 1)
    mean = s * inv
    h = lax.dot_general(mean, wn_ref[0], (((1,), (1,)), ((), ())),
                        preferred_element_type=jnp.float32)
    h = h + bn_ref[0]
    h = h + lax.dot_general(xd_ref[0], wr_ref[0], (((1,), (1,)), ((), ())),
                            preferred_element_type=jnp.float32)
    m = jnp.mean(h, axis=-1, keepdims=True)
    ctr = h - m
    v = jnp.mean(ctr * ctr, axis=-1, keepdims=True)
    y = ctr * lax.rsqrt(v + 1e-5) * g_ref[0] + b_ref[0]
    out_ref[0] = jnp.maximum(y, 0.0)


def _tc_dense(acc, cnts, xs, wn, bn, wr, g, b, flip_xd):
    xd_map = (lambda t, i: (1 - t, i, 0)) if flip_xd else \
             (lambda t, i: (t, i, 0))
    return pl.pallas_call(
        _dense_body,
        grid=(2, N_ACC // RB),
        in_specs=[
            pl.BlockSpec((1, RB, D), lambda t, i: (t, i, 0)),
            pl.BlockSpec((1, NS, N_ACC), lambda t, i: (t, 0, 0)),
            pl.BlockSpec((1, RB, D), xd_map),
            pl.BlockSpec((1, D, D), lambda t, i: (t, 0, 0)),
            pl.BlockSpec((1, 1, D), lambda t, i: (t, 0, 0)),
            pl.BlockSpec((1, D, D), lambda t, i: (t, 0, 0)),
            pl.BlockSpec((1, 1, D), lambda t, i: (t, 0, 0)),
            pl.BlockSpec((1, 1, D), lambda t, i: (t, 0, 0)),
        ],
        out_specs=pl.BlockSpec((1, RB, D), lambda t, i: (t, i, 0)),
        out_shape=jax.ShapeDtypeStruct((2, N_ACC, D), jnp.float32),
    )(acc, cnts, xs, wn, bn, wr, g, b)


N_CROWS_RAW = E_RAW // CHUNK
_PAD_CROWS = N_CROWS - N_CROWS_RAW


def _prep_body(eiu_ref, eii_ref, s0_ref, s1_ref, d_ref):
    izeros = jnp.zeros((_PAD_CROWS, CHUNK), jnp.int32)
    iacc = jnp.full((_PAD_CROWS, CHUNK), N_ACC, jnp.int32)
    ipadn = jnp.full((_PAD_CROWS, CHUNK), N_NODES, jnp.int32)
    s0_ref[0] = jnp.concatenate([eiu_ref[0], izeros], axis=0)
    s0_ref[1] = jnp.concatenate([eii_ref[0] + N_ACC, iacc], axis=0)
    s1_ref[0] = jnp.concatenate([eiu_ref[0] + N_ACC, iacc], axis=0)
    s1_ref[1] = jnp.concatenate([eii_ref[0], izeros], axis=0)
    d_ref[0] = jnp.concatenate([eiu_ref[1], ipadn], axis=0)
    d_ref[1] = jnp.concatenate([eii_ref[1], ipadn], axis=0)


def _prep_edges(ei_u2i, ei_i2u):
    eshape = jax.ShapeDtypeStruct((2, N_CROWS, CHUNK), jnp.int32)
    full = pl.BlockSpec((2, N_CROWS_RAW, CHUNK), lambda: (0, 0, 0))
    return pl.pallas_call(
        _prep_body,
        grid=(),
        in_specs=[full, full],
        out_specs=[pl.BlockSpec((2, N_CROWS, CHUNK), lambda: (0, 0, 0))] * 3,
        out_shape=[eshape, eshape, eshape],
    )(ei_u2i.reshape(2, N_CROWS_RAW, CHUNK),
      ei_i2u.reshape(2, N_CROWS_RAW, CHUNK))


def kernel(x_user, x_item, edge_index_user_item, edge_index_item_user,
           Wn_0_u2i, bn_0_u2i, Wr_0_u2i, Wn_0_i2u, bn_0_i2u, Wr_0_i2u,
           g_0_user, b_0_user, g_0_item, b_0_item,
           Wn_1_u2i, bn_1_u2i, Wr_1_u2i, Wn_1_i2u, bn_1_i2u, Wr_1_i2u,
           g_1_user, b_1_user, g_1_item, b_1_item):
    src_l0, src_l1, dst2 = _prep_edges(edge_index_user_item,
                                       edge_index_item_user)
    zeros_acc = jnp.zeros((N_ACC, D), jnp.float32)
    pad_rows = jnp.zeros((N_ACC - N_NODES, D), jnp.float32)
    xs0 = jnp.stack([jnp.concatenate([x_user, pad_rows]),
                     jnp.concatenate([x_item, pad_rows])])

    def stk(a, b_):
        return jnp.stack([a, b_])

    wn0 = stk(Wn_0_u2i, Wn_0_i2u)
    bn0 = stk(bn_0_u2i, bn_0_i2u).reshape(2, 1, D)
    wr0 = stk(Wr_0_u2i, Wr_0_i2u)
    g0 = stk(g_0_item, g_0_user).reshape(2, 1, D)
    b0 = stk(b_0_item, b_0_user).reshape(2, 1, D)
    wn1 = stk(Wn_1_u2i, Wn_1_i2u)
    bn1 = stk(bn_1_u2i, bn_1_i2u).reshape(2, 1, D)
    wr1 = stk(Wr_1_u2i, Wr_1_i2u)
    g1 = stk(g_1_item, g_1_user).reshape(2, 1, D)
    b1 = stk(b_1_item, b_1_user).reshape(2, 1, D)

    xs0f = xs0.reshape(2 * N_ACC, D)
    acc0, cnts = _sc_layer_cnt(xs0f, src_l0, dst2, zeros_acc)
    o1 = _tc_dense(acc0, cnts, xs0, wn0, bn0, wr0, g0, b0, flip_xd=True)

    acc1 = _sc_layer(o1.reshape(2 * N_ACC, D), src_l1, dst2, zeros_acc)
    o2 = _tc_dense(acc1, cnts, o1, wn1, bn1, wr1, g1, b1, flip_xd=False)

    return o2[1, :N_NODES], o2[0, :N_NODES]


# speedup vs baseline: 2.7436x; 2.7436x over previous
import functools

import jax
import jax.numpy as jnp
from jax import lax
from jax.experimental import pallas as pl
from jax.experimental.pallas import tpu as pltpu
from jax.experimental.pallas import tpu_sc as plsc

NC, NS, L = 2, 16, 16
D = 128
CHUNK = 128
N_NODES = 10000
N_ACC = 10240
ROWS_PER_TILE = N_ACC // NS
E_RAW = 320000
E_PAD = 32 * 80 * CHUNK
N_CROWS = E_PAD // CHUNK
TILE_CHUNKS = N_CROWS // NS
NBUF = 2
SBUF = 4

_MESH = plsc.VectorSubcoreMesh(core_axis_name="c", subcore_axis_name="s")
_SC_PARAMS = pltpu.CompilerParams(needs_layout_passes=False)


def _make_sc_layer(with_counts):
    if with_counts:
        out_type = [jax.ShapeDtypeStruct((NC, N_ACC, D), jnp.float32),
                    jax.ShapeDtypeStruct((NC, NS, N_ACC), jnp.float32)]
    else:
        out_type = jax.ShapeDtypeStruct((NC, N_ACC, D), jnp.float32)

    scratch = [
        pltpu.VMEM_SHARED((N_ACC, D), jnp.float32),
        pltpu.VMEM((SBUF, CHUNK), jnp.int32),
        pltpu.VMEM((NBUF, CHUNK), jnp.int32),
        [pltpu.VMEM((CHUNK, D), jnp.float32) for _ in range(NBUF)],
        [pltpu.SemaphoreType.DMA for _ in range(NBUF)],
        [pltpu.SemaphoreType.DMA for _ in range(NBUF)],
        [pltpu.SemaphoreType.DMA for _ in range(SBUF)],
    ]
    if with_counts:
        scratch.append(pltpu.VMEM((N_ACC,), jnp.float32))

    @functools.partial(
        pl.kernel, mesh=_MESH, out_type=out_type, scratch_types=scratch,
        compiler_params=_SC_PARAMS,
        name="sc_layer_cnt" if with_counts else "sc_layer",
    )
    def k(*args):
        if with_counts:
            (x_hbm, src_hbm, dst_hbm, zeros_hbm, acc_out, cnt_out,
             acc_sp, src_ring, dst_ring, rows, gsems, dsems, ssems,
             cnt_v) = args
        else:
            (x_hbm, src_hbm, dst_hbm, zeros_hbm, acc_out,
             acc_sp, src_ring, dst_ring, rows, gsems, dsems, ssems) = args
        cid = lax.axis_index("c")
        sid = lax.axis_index("s")
        row0 = sid * ROWS_PER_TILE
        crow0 = sid * TILE_CHUNKS

        pltpu.sync_copy(zeros_hbm.at[pl.ds(row0, ROWS_PER_TILE)],
                        acc_sp.at[pl.ds(row0, ROWS_PER_TILE)])
        if with_counts:
            def zbody(i, c):
                cnt_v[pl.ds(i * L, L)] = jnp.zeros((L,), jnp.float32)
                return c
            lax.fori_loop(0, N_ACC // L, zbody, 0)
        plsc.subcore_barrier()

        for s in range(NBUF):
            pltpu.sync_copy(src_hbm.at[cid, crow0 + s], src_ring.at[s])
        for s in range(NBUF, SBUF):
            pltpu.async_copy(src_hbm.at[cid, crow0 + s], src_ring.at[s],
                             ssems[s])
        for b in range(NBUF):
            pltpu.async_copy(dst_hbm.at[cid, crow0 + b], dst_ring.at[b],
                             dsems[b])
            pltpu.async_copy(x_hbm.at[src_ring.at[b]], rows[b], gsems[b])

        def outer(o, c):
            base = o * SBUF
            for s in range(SBUF):
                i = base + s
                b = s % NBUF
                pltpu.make_async_copy(dst_hbm.at[cid, crow0 + i],
                                      dst_ring.at[b], dsems[b]).wait()
                pltpu.make_async_copy(x_hbm.at[src_ring.at[s]], rows[b],
                                      gsems[b]).wait()
                pltpu.sync_copy(rows[b], acc_sp.at[dst_ring.at[b]], add=True)
                if with_counts:
                    def cbody(j, cc):
                        dv = dst_ring[b, pl.ds(j * L, L)]
                        plsc.addupdate_scatter(cnt_v, [dv],
                                               jnp.ones((L,), jnp.float32))
                        return cc
                    lax.fori_loop(0, CHUNK // L, cbody, 0)

                @pl.when(i + NBUF < TILE_CHUNKS)
                def _():
                    nxt = i + NBUF
                    sn = (s + NBUF) % SBUF
                    pltpu.make_async_copy(src_hbm.at[cid, crow0 + nxt],
                                          src_ring.at[sn], ssems[sn]).wait()
                    pltpu.async_copy(x_hbm.at[src_ring.at[sn]],
                                     rows[b], gsems[b])
                    pltpu.async_copy(dst_hbm.at[cid, crow0 + nxt],
                                     dst_ring.at[b], dsems[b])

                @pl.when(i + SBUF < TILE_CHUNKS)
                def _():
                    pltpu.async_copy(src_hbm.at[cid, crow0 + i + SBUF],
                                     src_ring.at[s], ssems[s])
            return c

        lax.fori_loop(0, TILE_CHUNKS // SBUF, outer, 0)
        plsc.subcore_barrier()

        pltpu.sync_copy(acc_sp.at[pl.ds(row0, ROWS_PER_TILE)],
                        acc_out.at[cid, pl.ds(row0, ROWS_PER_TILE)])
        if with_counts:
            pltpu.sync_copy(cnt_v, cnt_out.at[cid, sid])

    return k


_sc_layer_cnt = _make_sc_layer(True)
_sc_layer = _make_sc_layer(False)

RB = 1280


def _dense_body(acc_ref, cnt_ref, xd_ref, wn_ref, bn_ref, wr_ref,
                g_ref, b_ref, out_ref):
    i = pl.program_id(1)
    s = acc_ref[0]
    c = jnp.sum(cnt_ref[0, :, pl.ds(i * RB, RB)], axis=0)
    inv = (1.0 / jnp.maximum(c, 1.0)).reshape(RB, 1)
    mean = s * inv
    h = lax.dot_general(mean, wn_ref[0], (((1,), (1,)), ((), ())),
                        preferred_element_type=jnp.float32)
    h = h + bn_ref[0]
    h = h + lax.dot_general(xd_ref[0], wr_ref[0], (((1,), (1,)), ((), ())),
                            preferred_element_type=jnp.float32)
    m = jnp.mean(h, axis=-1, keepdims=True)
    ctr = h - m
    v = jnp.mean(ctr * ctr, axis=-1, keepdims=True)
    y = ctr * lax.rsqrt(v + 1e-5) * g_ref[0] + b_ref[0]
    out_ref[0] = jnp.maximum(y, 0.0)


def _tc_dense(acc, cnts, xs, wn, bn, wr, g, b, flip_xd):
    xd_map = (lambda t, i: (1 - t, i, 0)) if flip_xd else \
             (lambda t, i: (t, i, 0))
    return pl.pallas_call(
        _dense_body,
        grid=(2, N_ACC // RB),
        in_specs=[
            pl.BlockSpec((1, RB, D), lambda t, i: (t, i, 0)),
            pl.BlockSpec((1, NS, N_ACC), lambda t, i: (t, 0, 0)),
            pl.BlockSpec((1, RB, D), xd_map),
            pl.BlockSpec((1, D, D), lambda t, i: (t, 0, 0)),
            pl.BlockSpec((1, 1, D), lambda t, i: (t, 0, 0)),
            pl.BlockSpec((1, D, D), lambda t, i: (t, 0, 0)),
            pl.BlockSpec((1, 1, D), lambda t, i: (t, 0, 0)),
            pl.BlockSpec((1, 1, D), lambda t, i: (t, 0, 0)),
        ],
        out_specs=pl.BlockSpec((1, RB, D), lambda t, i: (t, i, 0)),
        out_shape=jax.ShapeDtypeStruct((2, N_ACC, D), jnp.float32),
    )(acc, cnts, xs, wn, bn, wr, g, b)


N_CROWS_RAW = E_RAW // CHUNK
_PAD_CROWS = N_CROWS - N_CROWS_RAW


def _prep_body(eiu_ref, eii_ref, s0_ref, s1_ref, d_ref):
    flat = (lax.broadcasted_iota(jnp.int32, (_PAD_CROWS, CHUNK), 0) * CHUNK
            + lax.broadcasted_iota(jnp.int32, (_PAD_CROWS, CHUNK), 1))
    spread = flat % 256
    izeros = spread
    iacc = spread + N_ACC
    ipadn = N_NODES + flat % (N_ACC - N_NODES)
    s0_ref[0] = jnp.concatenate([eiu_ref[0], izeros], axis=0)
    s0_ref[1] = jnp.concatenate([eii_ref[0] + N_ACC, iacc], axis=0)
    s1_ref[0] = jnp.concatenate([eiu_ref[0] + N_ACC, iacc], axis=0)
    s1_ref[1] = jnp.concatenate([eii_ref[0], izeros], axis=0)
    d_ref[0] = jnp.concatenate([eiu_ref[1], ipadn], axis=0)
    d_ref[1] = jnp.concatenate([eii_ref[1], ipadn], axis=0)


def _prep_edges(ei_u2i, ei_i2u):
    eshape = jax.ShapeDtypeStruct((2, N_CROWS, CHUNK), jnp.int32)
    full = pl.BlockSpec((2, N_CROWS_RAW, CHUNK), lambda: (0, 0, 0))
    return pl.pallas_call(
        _prep_body,
        grid=(),
        in_specs=[full, full],
        out_specs=[pl.BlockSpec((2, N_CROWS, CHUNK), lambda: (0, 0, 0))] * 3,
        out_shape=[eshape, eshape, eshape],
    )(ei_u2i.reshape(2, N_CROWS_RAW, CHUNK),
      ei_i2u.reshape(2, N_CROWS_RAW, CHUNK))


def kernel(x_user, x_item, edge_index_user_item, edge_index_item_user,
           Wn_0_u2i, bn_0_u2i, Wr_0_u2i, Wn_0_i2u, bn_0_i2u, Wr_0_i2u,
           g_0_user, b_0_user, g_0_item, b_0_item,
           Wn_1_u2i, bn_1_u2i, Wr_1_u2i, Wn_1_i2u, bn_1_i2u, Wr_1_i2u,
           g_1_user, b_1_user, g_1_item, b_1_item):
    src_l0, src_l1, dst2 = _prep_edges(edge_index_user_item,
                                       edge_index_item_user)
    zeros_acc = jnp.zeros((N_ACC, D), jnp.float32)
    pad_rows = jnp.zeros((N_ACC - N_NODES, D), jnp.float32)
    xs0 = jnp.stack([jnp.concatenate([x_user, pad_rows]),
                     jnp.concatenate([x_item, pad_rows])])

    def stk(a, b_):
        return jnp.stack([a, b_])

    wn0 = stk(Wn_0_u2i, Wn_0_i2u)
    bn0 = stk(bn_0_u2i, bn_0_i2u).reshape(2, 1, D)
    wr0 = stk(Wr_0_u2i, Wr_0_i2u)
    g0 = stk(g_0_item, g_0_user).reshape(2, 1, D)
    b0 = stk(b_0_item, b_0_user).reshape(2, 1, D)
    wn1 = stk(Wn_1_u2i, Wn_1_i2u)
    bn1 = stk(bn_1_u2i, bn_1_i2u).reshape(2, 1, D)
    wr1 = stk(Wr_1_u2i, Wr_1_i2u)
    g1 = stk(g_1_item, g_1_user).reshape(2, 1, D)
    b1 = stk(b_1_item, b_1_user).reshape(2, 1, D)

    xs0f = xs0.reshape(2 * N_ACC, D)
    acc0, cnts = _sc_layer_cnt(xs0f, src_l0, dst2, zeros_acc)
    o1 = _tc_dense(acc0, cnts, xs0, wn0, bn0, wr0, g0, b0, flip_xd=True)

    acc1 = _sc_layer(o1.reshape(2 * N_ACC, D), src_l1, dst2, zeros_acc)
    o2 = _tc_dense(acc1, cnts, o1, wn1, bn1, wr1, g1, b1, flip_xd=False)

    return o2[1, :N_NODES], o2[0, :N_NODES]
